# baseline (speedup 1.0000x reference)
import jax
import jax.numpy as jnp
from jax.experimental import pallas as pl
from jax.experimental.pallas import tpu as pltpu


def _pick_bm(n: int) -> int:
    for bm in (512, 400, 256, 200, 128, 80, 64, 40, 16, 8):
        if n % bm == 0:
            return bm
    return n


def _spmm_kernel(a_ref, x_ref, o_ref):
    o_ref[...] = jnp.dot(a_ref[...], x_ref[...],
                         preferred_element_type=jnp.float32)


def _spmm_proj_kernel(a_ref, y_ref, w_ref, o_ref):
    t = jnp.dot(a_ref[...], y_ref[...], preferred_element_type=jnp.float32)
    o_ref[...] = 0.25 * jnp.dot(t, w_ref[...],
                                preferred_element_type=jnp.float32)


def kernel(feature, adj_self, weight):
    n, f = feature.shape
    e = weight.shape[1]
    bm = _pick_bm(n)
    grid = (n // bm,)
    params = pltpu.CompilerParams(dimension_semantics=("parallel",))

    y = pl.pallas_call(
        _spmm_kernel,
        grid=grid,
        in_specs=[
            pl.BlockSpec((bm, n), lambda i: (i, 0)),
            pl.BlockSpec((n, f), lambda i: (0, 0)),
        ],
        out_specs=pl.BlockSpec((bm, f), lambda i: (i, 0)),
        out_shape=jax.ShapeDtypeStruct((n, f), jnp.float32),
        compiler_params=params,
    )(adj_self, feature)

    z = pl.pallas_call(
        _spmm_proj_kernel,
        grid=grid,
        in_specs=[
            pl.BlockSpec((bm, n), lambda i: (i, 0)),
            pl.BlockSpec((n, f), lambda i: (0, 0)),
            pl.BlockSpec((f, e), lambda i: (0, 0)),
        ],
        out_specs=pl.BlockSpec((bm, e), lambda i: (i, 0)),
        out_shape=jax.ShapeDtypeStruct((n, e), jnp.float32),
        compiler_params=params,
    )(adj_self, y, weight)
    return z
